# baseline (speedup 1.0000x reference)
import functools

import jax
import jax.numpy as jnp
from jax import lax
from jax.experimental import pallas as pl
from jax.experimental.pallas import tpu as pltpu
from jax.experimental.pallas import tpu_sc as plsc


_BATCH = 100
_ROW = 256000
_NC, _NS = 2, 16
_NW = _NC * _NS
_SLICE = _ROW // _NW
_VREGS = _SLICE // 16
_U = 50
_GPAD = 112
_RPS = 4


def _mesh():
    return plsc.VectorSubcoreMesh(core_axis_name="c", subcore_axis_name="s")


def _sc_partials(xf):
    @functools.partial(
        pl.kernel,
        mesh=_mesh(),
        out_type=jax.ShapeDtypeStruct((_NC, _NS, _GPAD, 16), jnp.float32),
        scratch_types=[
            pltpu.VMEM((_SLICE,), jnp.float32),
            pltpu.VMEM((_SLICE,), jnp.float32),
            pltpu.VMEM((_GPAD, 16), jnp.float32),
            pltpu.SemaphoreType.DMA,
            pltpu.SemaphoreType.DMA,
        ],
    )
    def ka(x_hbm, out_hbm, buf0, buf1, part2, sem0, sem1):
        cid = lax.axis_index("c")
        sid = lax.axis_index("s")
        off = (sid * _NC + cid) * _SLICE
        bufs = (buf0, buf1)
        sems = (sem0, sem1)

        zero16 = jnp.zeros((16,), jnp.float32)

        pltpu.async_copy(x_hbm.at[pl.ds(off, _SLICE)], buf0, sem0)
        pltpu.async_copy(x_hbm.at[pl.ds(_ROW + off, _SLICE)], buf1, sem1)

        for r in range(_BATCH, _GPAD):
            part2[r] = zero16

        def outer(i, carry):
            for b in range(2):
                g = 2 * i + b
                buf, sem = bufs[b], sems[b]
                pltpu.make_async_copy(
                    x_hbm.at[pl.ds(g * _ROW + off, _SLICE)], buf, sem
                ).wait()

                def inner(j, accs):
                    base = j * (_U * 16)
                    accs = list(accs)
                    for u in range(_U):
                        accs[u % 8] = accs[u % 8] + buf[pl.ds(base + u * 16, 16)]
                    return tuple(accs)

                accs = lax.fori_loop(0, _VREGS // _U, inner, (zero16,) * 8)
                acc = (
                    ((accs[0] + accs[1]) + (accs[2] + accs[3]))
                    + ((accs[4] + accs[5]) + (accs[6] + accs[7]))
                )
                part2[g] = acc

                @pl.when(g + 2 < _BATCH)
                def _():
                    pltpu.async_copy(
                        x_hbm.at[pl.ds((g + 2) * _ROW + off, _SLICE)], buf, sem
                    )

            return carry

        lax.fori_loop(0, _BATCH // 2, outer, 0)

        pltpu.sync_copy(part2, out_hbm.at[cid, sid])

    return ka(xf)


def _sc_combine(pf):
    @functools.partial(
        pl.kernel,
        mesh=_mesh(),
        out_type=jax.ShapeDtypeStruct((_NW * 16,), jnp.float32),
        scratch_types=[
            pltpu.VMEM((_NW, 16), jnp.float32),
            pltpu.VMEM((16,), jnp.float32),
            pltpu.SemaphoreType.DMA,
        ],
    )
    def kb(p_hbm, out_hbm, rowbuf, vbuf, sem):
        cid = lax.axis_index("c")
        sid = lax.axis_index("s")
        myid = cid * _NS + sid

        zero16 = jnp.zeros((16,), jnp.float32)
        lanes = lax.iota(jnp.int32, 16)

        @pl.when(myid < _GPAD // _RPS)
        def _():
            v = zero16
            for i in range(_RPS):
                r = myid * _RPS + i
                for t in range(_NW):
                    pltpu.async_copy(
                        p_hbm.at[pl.ds((t * _GPAD + r) * 16, 16)],
                        rowbuf.at[t], sem,
                    )
                for t in range(_NW):
                    pltpu.make_async_copy(
                        p_hbm.at[pl.ds((t * _GPAD + r) * 16, 16)],
                        rowbuf.at[t], sem,
                    ).wait()
                acc = rowbuf[0]
                for t in range(1, _NW):
                    acc = acc + rowbuf[t]
                e = [acc[l] for l in range(16)]
                for step in (8, 4, 2, 1):
                    e = [e[m] + e[m + step] for m in range(step)]
                v = jnp.where(lanes == i, e[0], v)
            vbuf[...] = v
            pltpu.sync_copy(vbuf, out_hbm.at[pl.ds(myid * 16, 16)])

    return kb(pf)


def kernel(x, batch):
    parts = _sc_partials(x.reshape(-1))
    packed = _sc_combine(parts.reshape(-1))
    tot = packed.reshape(_NW, 16)[: _GPAD // _RPS, :_RPS].reshape(_GPAD)
    return tot[:_BATCH].astype(x.dtype)


# speedup vs baseline: 2.3856x; 2.3856x over previous
import functools

import jax
import jax.numpy as jnp
from jax import lax
from jax.experimental import pallas as pl
from jax.experimental.pallas import tpu as pltpu
from jax.experimental.pallas import tpu_sc as plsc


_BATCH = 100
_D = 256
_NC, _NS = 2, 16
_NW = _NC * _NS
_GROWS = 200
_GSIZE = _GROWS * _D
_NGRAN = 100000 // _GROWS
_GPG = 1000 // _GROWS
_GPAD = 112
_RPS = 4


def _mesh():
    return plsc.VectorSubcoreMesh(core_axis_name="c", subcore_axis_name="s")


def _sc_partials(x):
    @functools.partial(
        pl.kernel,
        mesh=_mesh(),
        out_type=jax.ShapeDtypeStruct((_NW * _GPAD * 16,), jnp.float32),
        scratch_types=[
            pltpu.VMEM((_GROWS, _D), jnp.float32),
            pltpu.VMEM((_GROWS, _D), jnp.float32),
            pltpu.VMEM((_GPAD * 16,), jnp.float32),
            pltpu.SemaphoreType.DMA,
            pltpu.SemaphoreType.DMA,
        ],
    )
    def ka(x_hbm, out_hbm, buf0, buf1, part2, sem0, sem1):
        cid = lax.axis_index("c")
        sid = lax.axis_index("s")
        wid = cid * _NS + sid
        bufs = (buf0, buf1)
        sems = (sem0, sem1)

        zero16 = jnp.zeros((16,), jnp.float32)

        def gran_rows(gran):
            return pl.multiple_of(gran * _GROWS, 8)

        pltpu.async_copy(x_hbm.at[pl.ds(gran_rows(wid), _GROWS)], buf0, sem0)
        pltpu.async_copy(
            x_hbm.at[pl.ds(gran_rows(wid + _NW), _GROWS)], buf1, sem1
        )

        for r in range(_GPAD):
            part2[pl.ds(r * 16, 16)] = zero16

        def outer(i, carry):
            for b in range(2):
                c = 2 * i + b
                gran = wid + _NW * c
                buf, sem = bufs[b], sems[b]

                @pl.when(gran < _NGRAN)
                def _():
                    pltpu.make_async_copy(
                        x_hbm.at[pl.ds(gran_rows(gran), _GROWS)], buf, sem
                    ).wait()

                    def inner(j, accs):
                        accs = list(accs)
                        for rr in range(4):
                            row = 4 * j + rr
                            for l in range(16):
                                accs[(rr * 16 + l) % 8] = (
                                    accs[(rr * 16 + l) % 8]
                                    + buf[row, pl.ds(l * 16, 16)]
                                )
                        return tuple(accs)

                    accs = lax.fori_loop(
                        0, _GROWS // 4, inner, (zero16,) * 8
                    )
                    acc = (
                        ((accs[0] + accs[1]) + (accs[2] + accs[3]))
                        + ((accs[4] + accs[5]) + (accs[6] + accs[7]))
                    )
                    g = gran // _GPG
                    pv = part2[pl.ds(g * 16, 16)]
                    part2[pl.ds(g * 16, 16)] = pv + acc

                    gran2 = gran + 2 * _NW

                    @pl.when(gran2 < _NGRAN)
                    def _():
                        pltpu.async_copy(
                            x_hbm.at[pl.ds(gran_rows(gran2), _GROWS)],
                            buf, sem,
                        )

            return carry

        lax.fori_loop(0, 8, outer, 0)

        pltpu.sync_copy(part2, out_hbm.at[pl.ds(wid * _GPAD * 16, _GPAD * 16)])

    return ka(x)


def _sc_combine(pf):
    @functools.partial(
        pl.kernel,
        mesh=_mesh(),
        out_type=jax.ShapeDtypeStruct((_NW * 16,), jnp.float32),
        scratch_types=[
            pltpu.VMEM((_NW, 16), jnp.float32),
            pltpu.VMEM((16,), jnp.float32),
            pltpu.SemaphoreType.DMA,
        ],
    )
    def kb(p_hbm, out_hbm, rowbuf, vbuf, sem):
        cid = lax.axis_index("c")
        sid = lax.axis_index("s")
        myid = cid * _NS + sid

        zero16 = jnp.zeros((16,), jnp.float32)
        lanes = lax.iota(jnp.int32, 16)

        @pl.when(myid < _GPAD // _RPS)
        def _():
            v = zero16
            for i in range(_RPS):
                r = myid * _RPS + i
                for t in range(_NW):
                    pltpu.async_copy(
                        p_hbm.at[pl.ds((t * _GPAD + r) * 16, 16)],
                        rowbuf.at[t], sem,
                    )
                for t in range(_NW):
                    pltpu.make_async_copy(
                        p_hbm.at[pl.ds((t * _GPAD + r) * 16, 16)],
                        rowbuf.at[t], sem,
                    ).wait()
                acc = rowbuf[0]
                for t in range(1, _NW):
                    acc = acc + rowbuf[t]
                e = [acc[l] for l in range(16)]
                for step in (8, 4, 2, 1):
                    e = [e[m] + e[m + step] for m in range(step)]
                v = jnp.where(lanes == i, e[0], v)
            vbuf[...] = v
            pltpu.sync_copy(vbuf, out_hbm.at[pl.ds(myid * 16, 16)])

    return kb(pf)


def kernel(x, batch):
    parts = _sc_partials(x)
    packed = _sc_combine(parts)
    tot = packed.reshape(_NW, 16)[: _GPAD // _RPS, :_RPS].reshape(_GPAD)
    return tot[:_BATCH].astype(x.dtype)
